# baseline (speedup 1.0000x reference)
import functools
import math

import jax
import jax.numpy as jnp
from jax import lax
from jax.experimental import pallas as pl
from jax.experimental.pallas import tpu as pltpu
from jax.experimental.pallas import tpu_sc as plsc

_NS = 50000
_NA = 50000
_E = 800000
_H = 64
_L = 3
_QW = 16
_NQ = _H // _QW

_NC = 2
_NT = 16
_K = 128
_SEG_CH = 392
_PAD_E = _NT * _SEG_CH * _K
_OUT_N = 50048
_JUNK = _OUT_N
_ACC_ROWS = 50176
_ZROWS = _ACC_ROWS // _NT
_OROWS = _OUT_N // _NT
_TBL_N = 2 * _NQ * _NS

_f32 = jnp.float32
_mesh = plsc.VectorSubcoreMesh(core_axis_name="c", subcore_axis_name="s")
_sc_params = pltpu.CompilerParams(use_tc_tiling_on_sc=False)


@functools.partial(
    pl.kernel,
    out_type=jax.ShapeDtypeStruct((_NC * _OUT_N, _QW), _f32),
    mesh=_mesh,
    scratch_types=[
        pltpu.VMEM((_SEG_CH, _K), jnp.int32),
        pltpu.VMEM((_K,), jnp.int32),
        pltpu.VMEM((_K,), jnp.int32),
        pltpu.VMEM((8, 16), jnp.int32),
        pltpu.VMEM((_K, _QW), _f32),
        pltpu.VMEM_SHARED((_ACC_ROWS, _QW), _f32),
        pltpu.SemaphoreType.DMA,
    ],
    compiler_params=_sc_params,
)
def _seg_sum(x_hbm, edges_hbm, qoff_hbm, zeros_hbm, out_hbm,
             ed_v, src_v, dst_v, qoff_v, rows_v, acc, sem):
    c = lax.axis_index("c")
    t = lax.axis_index("s")
    pltpu.sync_copy(edges_hbm.at[t], ed_v)
    pltpu.sync_copy(qoff_hbm.at[c], qoff_v)
    pltpu.sync_copy(zeros_hbm, acc.at[pl.ds(t * _ZROWS, _ZROWS)])
    plsc.subcore_barrier()
    qoff = qoff_v[0, :]

    def body(j, carry):
        for u in range(_K // 16):
            p = ed_v[j, pl.ds(u * 16, 16)]
            src_v[pl.ds(u * 16, 16)] = (p & 0xFFFF) + qoff
            dst_v[pl.ds(u * 16, 16)] = lax.shift_right_logical(p, 16)
        pltpu.async_copy(x_hbm.at[src_v], rows_v, sem).wait()
        pltpu.sync_copy(rows_v, acc.at[dst_v], add=True)
        return carry

    lax.fori_loop(0, _SEG_CH, body, 0)
    plsc.subcore_barrier()
    pltpu.sync_copy(acc.at[pl.ds(t * _OROWS, _OROWS)],
                    out_hbm.at[pl.ds(c * _OUT_N + t * _OROWS, _OROWS)])


_BR = 1000
_GRID = _NS // _BR


def _transform_body(hs_ref, ha_ref, wl0_ref, wl1_ref, wr0_ref, wr1_ref,
                    x8_ref, ra_ref, rs_ref):
    hs = hs_ref[...]
    ha = ha_ref[...]
    xs = jnp.dot(hs, wl0_ref[...], preferred_element_type=_f32)
    xa = jnp.dot(ha, wl1_ref[...], preferred_element_type=_f32)
    ra_ref[...] = jnp.dot(ha, wr0_ref[...], preferred_element_type=_f32)
    rs_ref[...] = jnp.dot(hs, wr1_ref[...], preferred_element_type=_f32)
    for q in range(_NQ):
        x8_ref[q] = xs[:, q * _QW:(q + 1) * _QW]
        x8_ref[_NQ + q] = xa[:, q * _QW:(q + 1) * _QW]


_row_spec = pl.BlockSpec((_BR, _H), lambda i: (i, 0))
_w_spec = pl.BlockSpec((_H, _H), lambda i: (0, 0))
_x8_spec = pl.BlockSpec((2 * _NQ, _BR, _QW), lambda i: (0, i, 0))
_vec_spec = pl.BlockSpec((1, _H), lambda i: (0, 0))

_transform = pl.pallas_call(
    _transform_body,
    grid=(_GRID,),
    in_specs=[_row_spec, _row_spec, _w_spec, _w_spec, _w_spec, _w_spec],
    out_specs=[_x8_spec, _row_spec, _row_spec],
    out_shape=[
        jax.ShapeDtypeStruct((2 * _NQ, _NS, _QW), _f32),
        jax.ShapeDtypeStruct((_NA, _H), _f32),
        jax.ShapeDtypeStruct((_NS, _H), _f32),
    ],
)

_INV_SQRT2 = 1.0 / math.sqrt(2.0)


def _post_body(s_ref, cnt_ref, r_ref, bl_ref, g_ref, b_ref, out_ref):
    s = jnp.concatenate([s_ref[q] for q in range(_NQ)], axis=-1)
    cnt = cnt_ref[0][:, 0:1]
    inv = 1.0 / jnp.maximum(cnt, 1.0)
    x = s * inv + bl_ref[...] + r_ref[...]
    g = 0.5 * x * (1.0 + lax.erf(x * _INV_SQRT2))
    mu = jnp.mean(g, axis=-1, keepdims=True)
    var = jnp.mean((g - mu) ** 2, axis=-1, keepdims=True)
    out_ref[...] = (g - mu) / jnp.sqrt(var + 1e-5) * g_ref[...] + b_ref[...]


def _make_post(s_plane, cnt_plane):
    return pl.pallas_call(
        _post_body,
        grid=(_GRID,),
        in_specs=[
            pl.BlockSpec((_NQ, _BR, _QW), lambda i: (s_plane, i, 0)),
            pl.BlockSpec((1, _BR, _QW), lambda i: (cnt_plane, i, 0)),
            _row_spec, _vec_spec, _vec_spec, _vec_spec,
        ],
        out_specs=_row_spec,
        out_shape=jax.ShapeDtypeStruct((_NS, _H), _f32),
    )


_post_a = _make_post(0, 0)
_post_s = _make_post(1, 2)


def kernel(emb_s, emb_a, Wl, bl, Wr, gamma, beta,
           edge_index_s2a, edge_index_a2s):
    pad = _PAD_E - _E

    def prep(edge_index):
        src = edge_index[0].astype(jnp.int32)
        dst = edge_index[1].astype(jnp.int32)
        srcp = jnp.concatenate([src, jnp.zeros((pad,), jnp.int32)])
        dstp = jnp.concatenate([dst, jnp.full((pad,), _JUNK, jnp.int32)])
        packed = srcp | (dstp << 16)
        return packed.reshape(_NT, _SEG_CH, _K)

    ed_sa = prep(edge_index_s2a)
    ed_as = prep(edge_index_a2s)
    ed_all = jnp.stack([ed_sa, ed_sa, ed_as, ed_as])

    def mkoff(qa, qb):
        return jnp.stack([jnp.full((8, 16), qa * _NS, jnp.int32),
                          jnp.full((8, 16), qb * _NS, jnp.int32)])

    qoff_all = jnp.stack([mkoff(0, 1), mkoff(2, 3),
                          mkoff(4, 5), mkoff(6, 7)])
    qoff_cnt = mkoff(0, 0)
    zeros = jnp.zeros((_ZROWS, _QW), _f32)

    dyn0 = jnp.minimum(edge_index_s2a[0, 0], 0).astype(jnp.int32)

    ones_x = jnp.ones((_TBL_N, _QW), _f32)
    ed_cnt = jnp.stack([ed_sa & jnp.int32(-65536), ed_as & jnp.int32(-65536)])

    def cnt_body(carry):
        j, acc = carry
        ed_j = lax.dynamic_index_in_dim(ed_cnt, j, keepdims=False)
        out = _seg_sum(ones_x, ed_j, qoff_cnt, zeros)
        acc = lax.dynamic_update_slice(acc, out[None], (j, 0, 0))
        return j + 1, acc

    _, cnts = lax.while_loop(
        lambda c: c[0] < 2 + dyn0, cnt_body,
        (jnp.int32(0), jnp.zeros((2, _NC * _OUT_N, _QW), _f32)))
    cnts = cnts.reshape(2 * _NC, _OUT_N, _QW)

    WlT = jnp.swapaxes(Wl, -1, -2)
    WrT = jnp.swapaxes(Wr, -1, -2)

    def layer_body(carry):
        i, h_s, h_a = carry
        wlT = lax.dynamic_index_in_dim(WlT, i, keepdims=False)
        wrT = lax.dynamic_index_in_dim(WrT, i, keepdims=False)
        bl_i = lax.dynamic_index_in_dim(bl, i, keepdims=False)
        g_i = lax.dynamic_index_in_dim(gamma, i, keepdims=False)
        b_i = lax.dynamic_index_in_dim(beta, i, keepdims=False)
        x8, ra, rs = _transform(h_s, h_a, wlT[0], wlT[1], wrT[0], wrT[1])
        x8f = x8.reshape(_TBL_N, _QW)

        def seg_body(carry2):
            j, acc = carry2
            ed_j = lax.dynamic_index_in_dim(ed_all, j, keepdims=False)
            qoff_j = lax.dynamic_index_in_dim(qoff_all, j, keepdims=False)
            out = _seg_sum(x8f, ed_j, qoff_j, zeros)
            acc = lax.dynamic_update_slice(acc, out[None], (j, 0, 0))
            return j + 1, acc

        _, summs = lax.while_loop(
            lambda c: c[0] < 4 + dyn0, seg_body,
            (jnp.int32(0), jnp.zeros((4, _NC * _OUT_N, _QW), _f32)))
        summs = summs.reshape(4 * _NC, _OUT_N, _QW)
        gi = g_i.reshape(1, _H)
        bi = b_i.reshape(1, _H)
        h_a = _post_a(summs, cnts, ra, bl_i[0].reshape(1, _H), gi, bi)
        h_s = _post_s(summs, cnts, rs, bl_i[1].reshape(1, _H), gi, bi)
        return i + 1, h_s, h_a

    _, h_s, h_a = lax.while_loop(
        lambda c: c[0] < _L + dyn0, layer_body,
        (jnp.int32(0), emb_s, emb_a))
    return (h_s, h_a)


# speedup vs baseline: 3.9883x; 3.9883x over previous
import functools
import math

import jax
import jax.numpy as jnp
from jax import lax
from jax.experimental import pallas as pl
from jax.experimental.pallas import tpu as pltpu
from jax.experimental.pallas import tpu_sc as plsc

_NS = 50000
_NA = 50000
_E = 800000
_H = 64
_L = 3
_QW = 16
_NQ = _H // _QW

_NC = 2
_NT = 16
_K = 128
_SEG_CH = 400
_R = 4
_ROUNDS = _SEG_CH // _R
_PAD_E = _NT * _SEG_CH * _K
_OUT_N = 50048
_JUNK = _OUT_N
_ACC_ROWS = 50176
_ZROWS = _ACC_ROWS // _NT
_OROWS = _OUT_N // _NT
_TBL_N = (2 * _NQ + 1) * _NS

_f32 = jnp.float32
_mesh = plsc.VectorSubcoreMesh(core_axis_name="c", subcore_axis_name="s")
_sc_params = pltpu.CompilerParams(use_tc_tiling_on_sc=False)


@functools.partial(
    pl.kernel,
    out_type=jax.ShapeDtypeStruct((_NC * _OUT_N, _QW), _f32),
    mesh=_mesh,
    scratch_types=[
        pltpu.VMEM((_SEG_CH, _K), jnp.int32),
        pltpu.VMEM((2 * _R, _K), jnp.int32),
        pltpu.VMEM((2 * _R, _K), jnp.int32),
        pltpu.VMEM((8, 16), jnp.int32),
        pltpu.VMEM((2 * _R * _K, _QW), _f32),
        pltpu.VMEM_SHARED((_ACC_ROWS, _QW), _f32),
        pltpu.SemaphoreType.DMA((2 * _R,)),
        pltpu.SemaphoreType.DMA((2 * _R,)),
    ],
    compiler_params=_sc_params,
)
def _seg_sum(x_hbm, edges_hbm, qoff_hbm, zeros_hbm, out_hbm,
             ed_v, src_v, dst_v, qoff_v, rows_v, acc, gsem, ssem):
    c = lax.axis_index("c")
    t = lax.axis_index("s")
    pltpu.sync_copy(edges_hbm.at[t], ed_v)
    pltpu.sync_copy(qoff_hbm.at[c], qoff_v)
    pltpu.sync_copy(zeros_hbm, acc.at[pl.ds(t * _ZROWS, _ZROWS)])
    plsc.subcore_barrier()
    qoff = qoff_v[0, :]

    def prep_fire(j, g, r):
        s = g * _R + r
        for u in range(_K // 16):
            p = ed_v[j, pl.ds(u * 16, 16)]
            src_v[s, pl.ds(u * 16, 16)] = (p & 0xFFFF) + qoff
            dst_v[s, pl.ds(u * 16, 16)] = lax.shift_right_logical(p, 16)
        pltpu.async_copy(x_hbm.at[src_v.at[s]],
                         rows_v.at[pl.ds(s * _K, _K)], gsem.at[s])

    def wait_fire_scatter(g):
        for r in range(_R):
            s = g * _R + r
            pltpu.make_async_copy(x_hbm.at[src_v.at[s]],
                                  rows_v.at[pl.ds(s * _K, _K)],
                                  gsem.at[s]).wait()
        for r in range(_R):
            s = g * _R + r
            pltpu.async_copy(rows_v.at[pl.ds(s * _K, _K)],
                             acc.at[dst_v.at[s]], ssem.at[s], add=True)

    def drain_prep(g, m):
        for r in range(_R):
            s = g * _R + r
            pltpu.make_async_copy(rows_v.at[pl.ds(s * _K, _K)],
                                  acc.at[dst_v.at[s]], ssem.at[s]).wait()
        for r in range(_R):
            prep_fire(m * _R + r, g, r)

    for r in range(_R):
        prep_fire(r, 0, r)
    for r in range(_R):
        prep_fire(_R + r, 1, r)

    def rounds2(mm, carry):
        wait_fire_scatter(0)
        wait_fire_scatter(1)
        drain_prep(0, 2 * mm + 2)
        drain_prep(1, 2 * mm + 3)
        return carry

    lax.fori_loop(0, _ROUNDS // 2 - 1, rounds2, 0)
    wait_fire_scatter(0)
    wait_fire_scatter(1)
    for s in range(2 * _R):
        pltpu.make_async_copy(rows_v.at[pl.ds(s * _K, _K)],
                              acc.at[dst_v.at[s]], ssem.at[s]).wait()
    plsc.subcore_barrier()
    pltpu.sync_copy(acc.at[pl.ds(t * _OROWS, _OROWS)],
                    out_hbm.at[pl.ds(c * _OUT_N + t * _OROWS, _OROWS)])


_BR = 1000
_GRID = _NS // _BR


def _transform_body(hs_ref, ha_ref, wl0_ref, wl1_ref, wr0_ref, wr1_ref,
                    x8_ref, ra_ref, rs_ref):
    hs = hs_ref[...]
    ha = ha_ref[...]
    xs = jnp.dot(hs, wl0_ref[...], preferred_element_type=_f32)
    xa = jnp.dot(ha, wl1_ref[...], preferred_element_type=_f32)
    ra_ref[...] = jnp.dot(ha, wr0_ref[...], preferred_element_type=_f32)
    rs_ref[...] = jnp.dot(hs, wr1_ref[...], preferred_element_type=_f32)
    for q in range(_NQ):
        x8_ref[q] = xs[:, q * _QW:(q + 1) * _QW]
        x8_ref[_NQ + q] = xa[:, q * _QW:(q + 1) * _QW]
    x8_ref[2 * _NQ] = jnp.ones((_BR, _QW), _f32)


_row_spec = pl.BlockSpec((_BR, _H), lambda i: (i, 0))
_w_spec = pl.BlockSpec((_H, _H), lambda i: (0, 0))
_x8_spec = pl.BlockSpec((2 * _NQ + 1, _BR, _QW), lambda i: (0, i, 0))
_vec_spec = pl.BlockSpec((1, _H), lambda i: (0, 0))

_transform = pl.pallas_call(
    _transform_body,
    grid=(_GRID,),
    in_specs=[_row_spec, _row_spec, _w_spec, _w_spec, _w_spec, _w_spec],
    out_specs=[_x8_spec, _row_spec, _row_spec],
    out_shape=[
        jax.ShapeDtypeStruct((2 * _NQ + 1, _NS, _QW), _f32),
        jax.ShapeDtypeStruct((_NA, _H), _f32),
        jax.ShapeDtypeStruct((_NS, _H), _f32),
    ],
)

_INV_SQRT2 = 1.0 / math.sqrt(2.0)


def _post_body(s_ref, cnt_ref, r_ref, bl_ref, g_ref, b_ref, out_ref):
    s = jnp.concatenate([s_ref[q] for q in range(_NQ)], axis=-1)
    cnt = cnt_ref[0][:, 0:1]
    inv = 1.0 / jnp.maximum(cnt, 1.0)
    x = s * inv + bl_ref[...] + r_ref[...]
    g = 0.5 * x * (1.0 + lax.erf(x * _INV_SQRT2))
    mu = jnp.mean(g, axis=-1, keepdims=True)
    var = jnp.mean((g - mu) ** 2, axis=-1, keepdims=True)
    out_ref[...] = (g - mu) / jnp.sqrt(var + 1e-5) * g_ref[...] + b_ref[...]


def _make_post(s_plane, cnt_plane):
    return pl.pallas_call(
        _post_body,
        grid=(_GRID,),
        in_specs=[
            pl.BlockSpec((_NQ, _BR, _QW), lambda i: (s_plane, i, 0)),
            pl.BlockSpec((1, _BR, _QW), lambda i: (cnt_plane, i, 0)),
            _row_spec, _vec_spec, _vec_spec, _vec_spec,
        ],
        out_specs=_row_spec,
        out_shape=jax.ShapeDtypeStruct((_NS, _H), _f32),
    )


_post_a = _make_post(0, 8)
_post_s = _make_post(1, 10)


def kernel(emb_s, emb_a, Wl, bl, Wr, gamma, beta,
           edge_index_s2a, edge_index_a2s):
    pad = _PAD_E - _E

    def prep(edge_index):
        src = edge_index[0].astype(jnp.int32)
        dst = edge_index[1].astype(jnp.int32)
        srcp = jnp.concatenate([src, jnp.zeros((pad,), jnp.int32)])
        dstp = jnp.concatenate([dst, jnp.full((pad,), _JUNK, jnp.int32)])
        packed = srcp | (dstp << 16)
        return packed.reshape(_NT, _SEG_CH, _K)

    ed_sa = prep(edge_index_s2a)
    ed_as = prep(edge_index_a2s)
    ed_all = jnp.stack([ed_sa, ed_sa, ed_as, ed_as, ed_sa, ed_as])

    def mkoff(qa, qb):
        return jnp.stack([jnp.full((8, 16), qa * _NS, jnp.int32),
                          jnp.full((8, 16), qb * _NS, jnp.int32)])

    qoff_all = jnp.stack([mkoff(0, 1), mkoff(2, 3), mkoff(4, 5),
                          mkoff(6, 7), mkoff(8, 8), mkoff(8, 8)])
    zeros = jnp.zeros((_ZROWS, _QW), _f32)

    dyn0 = jnp.minimum(edge_index_s2a[0, 0], 0).astype(jnp.int32)

    WlT = jnp.swapaxes(Wl, -1, -2)
    WrT = jnp.swapaxes(Wr, -1, -2)

    def layer_body(carry):
        i, h_s, h_a, summs = carry
        wlT = lax.dynamic_index_in_dim(WlT, i, keepdims=False)
        wrT = lax.dynamic_index_in_dim(WrT, i, keepdims=False)
        bl_i = lax.dynamic_index_in_dim(bl, i, keepdims=False)
        g_i = lax.dynamic_index_in_dim(gamma, i, keepdims=False)
        b_i = lax.dynamic_index_in_dim(beta, i, keepdims=False)
        x8, ra, rs = _transform(h_s, h_a, wlT[0], wlT[1], wrT[0], wrT[1])
        x8f = x8.reshape(_TBL_N, _QW)
        npass = jnp.where(i == 0, 6, 4) + dyn0

        def seg_body(carry2):
            j, acc = carry2
            ed_j = lax.dynamic_index_in_dim(ed_all, j, keepdims=False)
            qoff_j = lax.dynamic_index_in_dim(qoff_all, j, keepdims=False)
            out = _seg_sum(x8f, ed_j, qoff_j, zeros)
            acc = lax.dynamic_update_slice(acc, out[None], (j, 0, 0))
            return j + 1, acc

        _, summs = lax.while_loop(lambda c: c[0] < npass, seg_body,
                                  (jnp.int32(0), summs))
        sp = summs.reshape(6 * _NC, _OUT_N, _QW)
        gi = g_i.reshape(1, _H)
        bi = b_i.reshape(1, _H)
        h_a = _post_a(sp, sp, ra, bl_i[0].reshape(1, _H), gi, bi)
        h_s = _post_s(sp, sp, rs, bl_i[1].reshape(1, _H), gi, bi)
        return i + 1, h_s, h_a, summs

    summs0 = jnp.zeros((6, _NC * _OUT_N, _QW), _f32)
    _, h_s, h_a, _ = lax.while_loop(
        lambda c: c[0] < _L + dyn0, layer_body,
        (jnp.int32(0), emb_s, emb_a, summs0))
    return (h_s, h_a)
